# baseline (speedup 1.0000x reference)
import functools

import jax
import jax.numpy as jnp
from jax import lax
from jax.experimental import pallas as pl
from jax.experimental.pallas import tpu as pltpu
from jax.experimental.pallas import tpu_sc as plsc

_F = 64
_VAR = 4096
_CON = 4096
_STEPS = 4
_BLK = 512
_NBLK = _VAR // _BLK


_STRIPES_PER_CORE = 8
_STRIPE_WORDS = 256 * _CON
_CHUNK = 8192
_NNZ_PAD = 1703936
_EDGES_PER_TILE = _NNZ_PAD // 16
_CHUNKS_PER_TILE = _EDGES_PER_TILE // _CHUNK
_WB_WORDS = _STRIPE_WORDS // 16


def _densify_body(row_hbm, col_hbm, val_hbm, out_hbm,
                  row_b, col_b, val_b, idx_b, sval_b, zbuf, stripe_sh):
    cid = lax.axis_index("c")
    sid = lax.axis_index("s")
    estart = sid * _EDGES_PER_TILE
    lanes = lax.iota(jnp.int32, 16)

    def zinit(i, _):
        zbuf[pl.ds(i * 16, 16)] = jnp.zeros((16,), jnp.float32)
        return 0
    lax.fori_loop(0, zbuf.shape[0] // 16, zinit, 0)

    def stripe_body(k, _):
        base = (cid * _STRIPES_PER_CORE + k) * _STRIPE_WORDS

        def zcopy(z, _):
            pltpu.sync_copy(
                zbuf, stripe_sh.at[pl.ds(sid * _WB_WORDS + z * zbuf.shape[0],
                                         zbuf.shape[0])])
            return 0
        lax.fori_loop(0, _WB_WORDS // zbuf.shape[0], zcopy, 0)
        plsc.subcore_barrier()

        def chunk_body(j, _):
            off = estart + j * _CHUNK
            pltpu.sync_copy(row_hbm.at[pl.ds(off, _CHUNK)], row_b)
            pltpu.sync_copy(col_hbm.at[pl.ds(off, _CHUNK)], col_b)
            pltpu.sync_copy(val_hbm.at[pl.ds(off, _CHUNK)], val_b)

            def vec_body(i, _):
                r = row_b[pl.ds(i * 16, 16)]
                c = col_b[pl.ds(i * 16, 16)]
                v = val_b[pl.ds(i * 16, 16)]
                local = r * _CON + c - base
                ok = (local >= 0) & (local < _STRIPE_WORDS)
                junk = i * 16 + lanes
                idx_b[pl.ds(i * 16, 16)] = jnp.where(ok, local, junk)
                sval_b[pl.ds(i * 16, 16)] = jnp.where(ok, v, 0.0)
                return 0
            lax.fori_loop(0, _CHUNK // 16, vec_body, 0)
            pltpu.sync_copy(sval_b, stripe_sh.at[idx_b], add=True)
            return 0
        lax.fori_loop(0, _CHUNKS_PER_TILE, chunk_body, 0)
        plsc.subcore_barrier()

        pltpu.sync_copy(stripe_sh.at[pl.ds(sid * _WB_WORDS, _WB_WORDS)],
                        out_hbm.at[pl.ds(base + sid * _WB_WORDS, _WB_WORDS)])
        plsc.subcore_barrier()
        return 0
    lax.fori_loop(0, _STRIPES_PER_CORE, stripe_body, 0)


def _sc_densify(row_p, col_p, val_p):
    fn = pl.kernel(
        _densify_body,
        out_type=jax.ShapeDtypeStruct((_VAR * _CON,), jnp.float32),
        mesh=plsc.VectorSubcoreMesh(core_axis_name="c", subcore_axis_name="s"),
        scratch_types=[
            pltpu.VMEM((_CHUNK,), jnp.int32),
            pltpu.VMEM((_CHUNK,), jnp.int32),
            pltpu.VMEM((_CHUNK,), jnp.float32),
            pltpu.VMEM((_CHUNK,), jnp.int32),
            pltpu.VMEM((_CHUNK,), jnp.float32),
            pltpu.VMEM((8192,), jnp.float32),
            pltpu.VMEM_SHARED((_STRIPE_WORDS,), jnp.float32),
        ],
    )
    return fn(row_p, col_p, val_p)


def _pair_norm(x):
    x = x - jnp.mean(x, axis=0, keepdims=True)
    rownorm_mean = jnp.sqrt(1e-06 + jnp.mean(jnp.sum(x * x, axis=1)))
    return x / rownorm_mean


def _leaky(x):
    return jnp.where(x >= 0, x, 0.01 * x)


def _mp_body(A_hbm, cond, noise,
             Wp1, bp1, Wp2, bp2, Wc1, bc1, Wc2, bc2,
             Wv1, bv1, Wv2, bv2, Wo1, bo1, Wo2, bo2,
             o0, o1, o2, o3, ablk, sem):
    outs = (o0, o1, o2, o3)

    def load_blk(b):
        cp = pltpu.make_async_copy(A_hbm.at[pl.ds(b * _BLK, _BLK), :], ablk, sem)
        cp.start()
        cp.wait()
        return ablk[...]

    h = _leaky(cond[...] * Wp1[...][0:1, :] + bp1[...][0:1, :])
    emb = _pair_norm(jnp.dot(h, Wp2[...], preferred_element_type=jnp.float32)
                     + bp2[...][0:1, :])

    constraints = emb
    variables = jnp.ones((_VAR, _F), dtype=jnp.float32)

    Wc1r = Wc1[...]
    cbias = (jnp.dot(emb, Wc1r[_F:2 * _F, :], preferred_element_type=jnp.float32)
             + bc1[...][0:1, :])

    for i in range(_STEPS):
        v2c = jnp.zeros((_CON, _F), dtype=jnp.float32)
        for b in range(_NBLK):
            a = load_blk(b)
            v2c = v2c + lax.dot_general(
                a, variables[b * _BLK:(b + 1) * _BLK, :],
                dimension_numbers=(((0,), (0,)), ((), ())),
                preferred_element_type=jnp.float32)
        hc = _leaky(jnp.dot(constraints, Wc1r[0:_F, :], preferred_element_type=jnp.float32)
                    + jnp.dot(v2c, Wc1r[2 * _F:3 * _F, :], preferred_element_type=jnp.float32)
                    + cbias)
        constraints = _pair_norm(jnp.dot(hc, Wc2[...], preferred_element_type=jnp.float32)
                                 + bc2[...][0:1, :])

        c2v_rows = []
        for b in range(_NBLK):
            a = load_blk(b)
            c2v_rows.append(jnp.dot(a, constraints, preferred_element_type=jnp.float32))
        c2v = jnp.concatenate(c2v_rows, axis=0)
        hv = _leaky(jnp.dot(variables, Wv1[...][0:_F, :], preferred_element_type=jnp.float32)
                    + jnp.dot(c2v, Wv1[...][_F:2 * _F, :], preferred_element_type=jnp.float32)
                    + bv1[...][0:1, :])
        variables = _pair_norm(jnp.dot(hv, Wv2[...], preferred_element_type=jnp.float32)
                               + bv2[...][0:1, :])

        ho = _leaky(jnp.dot(variables, Wo1[...], preferred_element_type=jnp.float32)
                    + bo1[...][0:1, :])
        out = jnp.sum(ho * Wo2[...][:, 0][None, :], axis=1, keepdims=True) + bo2[...][0, 0]
        logits = out + noise[...][i]
        outs[i][...] = 1.0 / (1.0 + jnp.exp(-logits))


def _message_passing(A, cond2d, noise, weights):
    out_shape = [jax.ShapeDtypeStruct((_VAR, 1), jnp.float32)] * _STEPS
    fn = pl.pallas_call(
        _mp_body,
        in_specs=[pl.BlockSpec(memory_space=pl.ANY)]
                 + [pl.BlockSpec(memory_space=pltpu.VMEM)] * (2 + len(weights)),
        out_specs=[pl.BlockSpec(memory_space=pltpu.VMEM)] * _STEPS,
        out_shape=out_shape,
        scratch_shapes=[pltpu.VMEM((_BLK, _CON), jnp.float32),
                        pltpu.SemaphoreType.DMA],
    )
    return fn(A, cond2d, noise, *weights)


def kernel(row_idx, col_idx, edge_vals, conditions_values,
           Wp1, bp1, Wp2, bp2, Wc1, bc1, Wc2, bc2,
           Wv1, bv1, Wv2, bv2, Wo1, bo1, Wo2, bo2):
    pad = _NNZ_PAD - row_idx.shape[0]
    row_p = jnp.pad(row_idx.astype(jnp.int32), (0, pad))
    col_p = jnp.pad(col_idx.astype(jnp.int32), (0, pad))
    val_p = jnp.pad(edge_vals, (0, pad))
    A = _sc_densify(row_p, col_p, val_p).reshape(_VAR, _CON)

    nkey = jax.random.key(42)
    noise = jnp.stack([
        3.0 * jax.random.normal(jax.random.fold_in(nkey, i), (_VAR, 1), dtype=jnp.float32)
        for i in range(_STEPS)])

    weights = (Wp1, bp1.reshape(1, _F), Wp2, bp2.reshape(1, _F),
               Wc1, bc1.reshape(1, _F), Wc2, bc2.reshape(1, _F),
               Wv1, bv1.reshape(1, _F), Wv2, bv2.reshape(1, _F),
               Wo1, bo1.reshape(1, _F), Wo2, bo2.reshape(1, 1))
    outs = _message_passing(A, conditions_values.reshape(_CON, 1), noise, weights)
    return tuple(outs)


# speedup vs baseline: 1.1875x; 1.1875x over previous
import functools

import jax
import jax.numpy as jnp
from jax import lax
from jax.experimental import pallas as pl
from jax.experimental.pallas import tpu as pltpu
from jax.experimental.pallas import tpu_sc as plsc

_F = 64
_VAR = 4096
_CON = 4096
_STEPS = 4
_BLK = 256
_NBLK = _VAR // _BLK


_STRIPES_PER_CORE = 8
_STRIPE_ROWS = 256
_STRIPE_WORDS = _STRIPE_ROWS * _CON
_LAST_ROWS = _VAR - (2 * _STRIPES_PER_CORE - 1) * _STRIPE_ROWS
_LAST_WORDS = _LAST_ROWS * _CON
_CHUNK = 8192
_NNZ_PAD = 1703936
_EDGES_PER_TILE = _NNZ_PAD // 16
_CHUNKS_PER_TILE = _EDGES_PER_TILE // _CHUNK
_WB_WORDS = _STRIPE_WORDS // 16
_WBL_WORDS = _LAST_WORDS // 16


def _densify_body(row_hbm, col_hbm, val_hbm, out_hbm,
                  row_b, col_b, val_b, idx_b, sval_b, zbuf, stripe_sh):
    cid = lax.axis_index("c")
    sid = lax.axis_index("s")
    estart = sid * _EDGES_PER_TILE
    lanes = lax.iota(jnp.int32, 16)

    def zinit(i, _):
        zbuf[pl.ds(i * 16, 16)] = jnp.zeros((16,), jnp.float32)
        return 0
    lax.fori_loop(0, zbuf.shape[0] // 16, zinit, 0)

    nzero = _STRIPE_WORDS // 16 // zbuf.shape[0]
    ztail = _STRIPE_WORDS // 16 - nzero * zbuf.shape[0]

    for k in range(_STRIPES_PER_CORE):
        stripe = cid * _STRIPES_PER_CORE + k
        base = stripe * _STRIPE_WORDS

        zoff = sid * (_STRIPE_WORDS // 16)

        def zcopy(z, _):
            pltpu.sync_copy(
                zbuf, stripe_sh.at[pl.ds(zoff + z * zbuf.shape[0],
                                         zbuf.shape[0])])
            return 0
        lax.fori_loop(0, nzero, zcopy, 0)
        if ztail:
            pltpu.sync_copy(zbuf.at[pl.ds(0, ztail)],
                            stripe_sh.at[pl.ds(zoff + nzero * zbuf.shape[0],
                                               ztail)])
        plsc.subcore_barrier()

        def chunk_body(j, _):
            off = estart + j * _CHUNK
            pltpu.sync_copy(row_hbm.at[pl.ds(off, _CHUNK)], row_b)
            pltpu.sync_copy(col_hbm.at[pl.ds(off, _CHUNK)], col_b)
            pltpu.sync_copy(val_hbm.at[pl.ds(off, _CHUNK)], val_b)

            def vec_body(i, _):
                r = row_b[pl.ds(i * 16, 16)]
                c = col_b[pl.ds(i * 16, 16)]
                v = val_b[pl.ds(i * 16, 16)]
                local = r * _CON + c - base
                ok = (local >= 0) & (local < _STRIPE_WORDS)
                junk = sid * _CHUNK + i * 16 + lanes
                idx_b[pl.ds(i * 16, 16)] = jnp.where(ok, local, junk)
                sval_b[pl.ds(i * 16, 16)] = jnp.where(ok, v, 0.0)
                return 0
            lax.fori_loop(0, _CHUNK // 16, vec_body, 0)
            pltpu.sync_copy(sval_b, stripe_sh.at[idx_b], add=True)
            return 0
        lax.fori_loop(0, _CHUNKS_PER_TILE, chunk_body, 0)
        plsc.subcore_barrier()

        if k < _STRIPES_PER_CORE - 1:
            pltpu.sync_copy(stripe_sh.at[pl.ds(sid * _WB_WORDS, _WB_WORDS)],
                            out_hbm.at[pl.ds(base + sid * _WB_WORDS, _WB_WORDS)])
        else:
            @pl.when(cid == 0)
            def _():
                pltpu.sync_copy(
                    stripe_sh.at[pl.ds(sid * _WB_WORDS, _WB_WORDS)],
                    out_hbm.at[pl.ds(base + sid * _WB_WORDS, _WB_WORDS)])

            @pl.when(cid == 1)
            def _():
                pltpu.sync_copy(
                    stripe_sh.at[pl.ds(sid * _WBL_WORDS, _WBL_WORDS)],
                    out_hbm.at[pl.ds(base + sid * _WBL_WORDS, _WBL_WORDS)])
        plsc.subcore_barrier()


def _sc_densify(row_p, col_p, val_p):
    fn = pl.kernel(
        _densify_body,
        out_type=jax.ShapeDtypeStruct((_VAR * _CON,), jnp.float32),
        mesh=plsc.VectorSubcoreMesh(core_axis_name="c", subcore_axis_name="s"),
        scratch_types=[
            pltpu.VMEM((_CHUNK,), jnp.int32),
            pltpu.VMEM((_CHUNK,), jnp.int32),
            pltpu.VMEM((_CHUNK,), jnp.float32),
            pltpu.VMEM((_CHUNK,), jnp.int32),
            pltpu.VMEM((_CHUNK,), jnp.float32),
            pltpu.VMEM((8192,), jnp.float32),
            pltpu.VMEM_SHARED((_STRIPE_WORDS,), jnp.float32),
        ],
    )
    return fn(row_p, col_p, val_p)


def _pair_norm(x):
    x = x - jnp.mean(x, axis=0, keepdims=True)
    rownorm_mean = jnp.sqrt(1e-06 + jnp.mean(jnp.sum(x * x, axis=1)))
    return x / rownorm_mean


def _leaky(x):
    return jnp.where(x >= 0, x, 0.01 * x)


def _mp_body(A_hbm, cond, noise,
             Wp1, bp1, Wp2, bp2, Wc1, bc1, Wc2, bc2,
             Wv1, bv1, Wv2, bv2, Wo1, bo1, Wo2, bo2,
             o0, o1, o2, o3, ablk0, ablk1, sem0, sem1):
    outs = (o0, o1, o2, o3)
    bufs = (ablk0, ablk1)
    sems = (sem0, sem1)

    def a_pass(consume):
        pltpu.make_async_copy(A_hbm.at[pl.ds(0, _BLK), :], bufs[0],
                              sems[0]).start()
        for b in range(_NBLK):
            pltpu.make_async_copy(A_hbm.at[pl.ds(b * _BLK, _BLK), :],
                                  bufs[b % 2], sems[b % 2]).wait()
            if b + 1 < _NBLK:
                pltpu.make_async_copy(A_hbm.at[pl.ds((b + 1) * _BLK, _BLK), :],
                                      bufs[(b + 1) % 2], sems[(b + 1) % 2]).start()
            consume(b, bufs[b % 2][...])

    h = _leaky(cond[...] * Wp1[...][0:1, :] + bp1[...][0:1, :])
    emb = _pair_norm(jnp.dot(h, Wp2[...], preferred_element_type=jnp.float32)
                     + bp2[...][0:1, :])

    constraints = emb
    variables = jnp.ones((_VAR, _F), dtype=jnp.float32)

    Wc1r = Wc1[...]
    cbias = (jnp.dot(emb, Wc1r[_F:2 * _F, :], preferred_element_type=jnp.float32)
             + bc1[...][0:1, :])

    for i in range(_STEPS):
        acc = [jnp.zeros((_CON, _F), dtype=jnp.float32)]

        def v2c_blk(b, a, variables=variables, acc=acc):
            acc[0] = acc[0] + lax.dot_general(
                a, variables[b * _BLK:(b + 1) * _BLK, :],
                dimension_numbers=(((0,), (0,)), ((), ())),
                preferred_element_type=jnp.float32)
        a_pass(v2c_blk)
        v2c = acc[0]
        hc = _leaky(jnp.dot(constraints, Wc1r[0:_F, :], preferred_element_type=jnp.float32)
                    + jnp.dot(v2c, Wc1r[2 * _F:3 * _F, :], preferred_element_type=jnp.float32)
                    + cbias)
        constraints = _pair_norm(jnp.dot(hc, Wc2[...], preferred_element_type=jnp.float32)
                                 + bc2[...][0:1, :])

        c2v_rows = [None] * _NBLK

        def c2v_blk(b, a, constraints=constraints, c2v_rows=c2v_rows):
            c2v_rows[b] = jnp.dot(a, constraints, preferred_element_type=jnp.float32)
        a_pass(c2v_blk)
        c2v = jnp.concatenate(c2v_rows, axis=0)
        hv = _leaky(jnp.dot(variables, Wv1[...][0:_F, :], preferred_element_type=jnp.float32)
                    + jnp.dot(c2v, Wv1[...][_F:2 * _F, :], preferred_element_type=jnp.float32)
                    + bv1[...][0:1, :])
        variables = _pair_norm(jnp.dot(hv, Wv2[...], preferred_element_type=jnp.float32)
                               + bv2[...][0:1, :])

        ho = _leaky(jnp.dot(variables, Wo1[...], preferred_element_type=jnp.float32)
                    + bo1[...][0:1, :])
        out = jnp.sum(ho * Wo2[...][:, 0][None, :], axis=1, keepdims=True) + bo2[...][0, 0]
        logits = out + noise[...][i]
        outs[i][...] = 1.0 / (1.0 + jnp.exp(-logits))


def _message_passing(A, cond2d, noise, weights):
    out_shape = [jax.ShapeDtypeStruct((_VAR, 1), jnp.float32)] * _STEPS
    fn = pl.pallas_call(
        _mp_body,
        in_specs=[pl.BlockSpec(memory_space=pl.ANY)]
                 + [pl.BlockSpec(memory_space=pltpu.VMEM)] * (2 + len(weights)),
        out_specs=[pl.BlockSpec(memory_space=pltpu.VMEM)] * _STEPS,
        out_shape=out_shape,
        scratch_shapes=[pltpu.VMEM((_BLK, _CON), jnp.float32),
                        pltpu.VMEM((_BLK, _CON), jnp.float32),
                        pltpu.SemaphoreType.DMA,
                        pltpu.SemaphoreType.DMA],
    )
    return fn(A, cond2d, noise, *weights)


def kernel(row_idx, col_idx, edge_vals, conditions_values,
           Wp1, bp1, Wp2, bp2, Wc1, bc1, Wc2, bc2,
           Wv1, bv1, Wv2, bv2, Wo1, bo1, Wo2, bo2):
    pad = _NNZ_PAD - row_idx.shape[0]
    row_p = jnp.pad(row_idx.astype(jnp.int32), (0, pad))
    col_p = jnp.pad(col_idx.astype(jnp.int32), (0, pad))
    val_p = jnp.pad(edge_vals, (0, pad))
    A = _sc_densify(row_p, col_p, val_p).reshape(_VAR, _CON)

    nkey = jax.random.key(42)
    noise = jnp.stack([
        3.0 * jax.random.normal(jax.random.fold_in(nkey, i), (_VAR, 1), dtype=jnp.float32)
        for i in range(_STEPS)])

    weights = (Wp1, bp1.reshape(1, _F), Wp2, bp2.reshape(1, _F),
               Wc1, bc1.reshape(1, _F), Wc2, bc2.reshape(1, _F),
               Wv1, bv1.reshape(1, _F), Wv2, bv2.reshape(1, _F),
               Wo1, bo1.reshape(1, _F), Wo2, bo2.reshape(1, 1))
    outs = _message_passing(A, conditions_values.reshape(_CON, 1), noise, weights)
    return tuple(outs)


# speedup vs baseline: 1.8873x; 1.5893x over previous
import functools

import jax
import jax.numpy as jnp
from jax import lax
from jax.experimental import pallas as pl
from jax.experimental.pallas import tpu as pltpu
from jax.experimental.pallas import tpu_sc as plsc

_F = 64
_VAR = 4096
_CON = 4096
_STEPS = 4
_BLK = 256
_NBLK = _VAR // _BLK


_STRIPES_PER_CORE = 8
_STRIPE_ROWS = 256
_STRIPE_WORDS = _STRIPE_ROWS * _CON
_LAST_ROWS = _VAR - (2 * _STRIPES_PER_CORE - 1) * _STRIPE_ROWS
_LAST_WORDS = _LAST_ROWS * _CON
_CHUNK = 4096
_NNZ_PAD = 1703936
_EDGES_PER_TILE = _NNZ_PAD // 16
_CHUNKS_PER_TILE = _EDGES_PER_TILE // _CHUNK
_WB_WORDS = _STRIPE_WORDS // 16
_WBL_WORDS = _LAST_WORDS // 16


def _densify_body(row_hbm, col_hbm, val_hbm, out_hbm,
                  row_b0, col_b0, val_b0, idx_b0, sval_b0,
                  row_b1, col_b1, val_b1, idx_b1, sval_b1,
                  zbuf, stripe_sh, sem_in0, sem_in1, sem_sc0, sem_sc1):
    cid = lax.axis_index("c")
    sid = lax.axis_index("s")
    estart = sid * _EDGES_PER_TILE
    lanes = lax.iota(jnp.int32, 16)
    rowb = (row_b0, row_b1)
    colb = (col_b0, col_b1)
    valb = (val_b0, val_b1)
    idxb = (idx_b0, idx_b1)
    svalb = (sval_b0, sval_b1)
    sem_in = (sem_in0, sem_in1)
    sem_sc = (sem_sc0, sem_sc1)

    def zinit(i, _):
        zbuf[pl.ds(i * 16, 16)] = jnp.zeros((16,), jnp.float32)
        return 0
    lax.fori_loop(0, zbuf.shape[0] // 16, zinit, 0)

    nzero = _STRIPE_WORDS // 16 // zbuf.shape[0]
    ztail = _STRIPE_WORDS // 16 - nzero * zbuf.shape[0]

    for k in range(_STRIPES_PER_CORE):
        stripe = cid * _STRIPES_PER_CORE + k
        base = stripe * _STRIPE_WORDS

        zoff = sid * (_STRIPE_WORDS // 16)

        def zcopy(z, _):
            pltpu.sync_copy(
                zbuf, stripe_sh.at[pl.ds(zoff + z * zbuf.shape[0],
                                         zbuf.shape[0])])
            return 0
        lax.fori_loop(0, nzero, zcopy, 0)
        if ztail:
            pltpu.sync_copy(zbuf.at[pl.ds(0, ztail)],
                            stripe_sh.at[pl.ds(zoff + nzero * zbuf.shape[0],
                                               ztail)])
        plsc.subcore_barrier()

        def start_in(j, s):
            off = estart + j * _CHUNK
            pltpu.async_copy(row_hbm.at[pl.ds(off, _CHUNK)], rowb[s], sem_in[s])
            pltpu.async_copy(col_hbm.at[pl.ds(off, _CHUNK)], colb[s], sem_in[s])
            pltpu.async_copy(val_hbm.at[pl.ds(off, _CHUNK)], valb[s], sem_in[s])

        def wait_in(s):
            for dst in (rowb[s], colb[s], valb[s]):
                pltpu.make_async_copy(row_hbm.at[pl.ds(0, _CHUNK)], dst,
                                      sem_in[s]).wait()

        def wait_sc(s):
            pltpu.make_async_copy(svalb[s], stripe_sh.at[idxb[s]],
                                  sem_sc[s]).wait()

        def process(jj, s):
            def vec_body(i, _):
                r = rowb[s][pl.ds(i * 16, 16)]
                c = colb[s][pl.ds(i * 16, 16)]
                v = valb[s][pl.ds(i * 16, 16)]
                local = r * _CON + c - base
                ok = (local >= 0) & (local < _STRIPE_WORDS)
                junk = sid * _CHUNK + i * 16 + lanes
                idxb[s][pl.ds(i * 16, 16)] = jnp.where(ok, local, junk)
                svalb[s][pl.ds(i * 16, 16)] = jnp.where(ok, v, 0.0)
                return 0
            lax.fori_loop(0, _CHUNK // 16, vec_body, 0)
            pltpu.async_copy(svalb[s], stripe_sh.at[idxb[s]],
                             sem_sc[s], add=True)

        start_in(0, 0)

        def pair_body(p, _):
            for s in (0, 1):
                jj = 2 * p + s

                @pl.when(jj + 1 < _CHUNKS_PER_TILE)
                def _():
                    start_in(jj + 1, (s + 1) % 2)
                wait_in(s)

                @pl.when(jj >= 2)
                def _():
                    wait_sc(s)
                process(jj, s)
            return 0
        lax.fori_loop(0, _CHUNKS_PER_TILE // 2, pair_body, 0)
        wait_sc(0)
        wait_sc(1)
        plsc.subcore_barrier()

        if k < _STRIPES_PER_CORE - 1:
            pltpu.sync_copy(stripe_sh.at[pl.ds(sid * _WB_WORDS, _WB_WORDS)],
                            out_hbm.at[pl.ds(base + sid * _WB_WORDS, _WB_WORDS)])
        else:
            @pl.when(cid == 0)
            def _():
                pltpu.sync_copy(
                    stripe_sh.at[pl.ds(sid * _WB_WORDS, _WB_WORDS)],
                    out_hbm.at[pl.ds(base + sid * _WB_WORDS, _WB_WORDS)])

            @pl.when(cid == 1)
            def _():
                pltpu.sync_copy(
                    stripe_sh.at[pl.ds(sid * _WBL_WORDS, _WBL_WORDS)],
                    out_hbm.at[pl.ds(base + sid * _WBL_WORDS, _WBL_WORDS)])
        plsc.subcore_barrier()


def _sc_densify(row_p, col_p, val_p):
    fn = pl.kernel(
        _densify_body,
        out_type=jax.ShapeDtypeStruct((_VAR * _CON,), jnp.float32),
        mesh=plsc.VectorSubcoreMesh(core_axis_name="c", subcore_axis_name="s"),
        scratch_types=[
            pltpu.VMEM((_CHUNK,), jnp.int32),
            pltpu.VMEM((_CHUNK,), jnp.int32),
            pltpu.VMEM((_CHUNK,), jnp.float32),
            pltpu.VMEM((_CHUNK,), jnp.int32),
            pltpu.VMEM((_CHUNK,), jnp.float32),
            pltpu.VMEM((_CHUNK,), jnp.int32),
            pltpu.VMEM((_CHUNK,), jnp.int32),
            pltpu.VMEM((_CHUNK,), jnp.float32),
            pltpu.VMEM((_CHUNK,), jnp.int32),
            pltpu.VMEM((_CHUNK,), jnp.float32),
            pltpu.VMEM((4096,), jnp.float32),
            pltpu.VMEM_SHARED((_STRIPE_WORDS,), jnp.float32),
            pltpu.SemaphoreType.DMA,
            pltpu.SemaphoreType.DMA,
            pltpu.SemaphoreType.DMA,
            pltpu.SemaphoreType.DMA,
        ],
    )
    return fn(row_p, col_p, val_p)


def _pair_norm(x):
    x = x - jnp.mean(x, axis=0, keepdims=True)
    rownorm_mean = jnp.sqrt(1e-06 + jnp.mean(jnp.sum(x * x, axis=1)))
    return x / rownorm_mean


def _leaky(x):
    return jnp.where(x >= 0, x, 0.01 * x)


def _mp_body(A_hbm, cond, noise,
             Wp1, bp1, Wp2, bp2, Wc1, bc1, Wc2, bc2,
             Wv1, bv1, Wv2, bv2, Wo1, bo1, Wo2, bo2,
             o0, o1, o2, o3, ablk0, ablk1, sem0, sem1):
    outs = (o0, o1, o2, o3)
    bufs = (ablk0, ablk1)
    sems = (sem0, sem1)

    def a_pass(consume):
        pltpu.make_async_copy(A_hbm.at[pl.ds(0, _BLK), :], bufs[0],
                              sems[0]).start()
        for b in range(_NBLK):
            pltpu.make_async_copy(A_hbm.at[pl.ds(b * _BLK, _BLK), :],
                                  bufs[b % 2], sems[b % 2]).wait()
            if b + 1 < _NBLK:
                pltpu.make_async_copy(A_hbm.at[pl.ds((b + 1) * _BLK, _BLK), :],
                                      bufs[(b + 1) % 2], sems[(b + 1) % 2]).start()
            consume(b, bufs[b % 2][...])

    h = _leaky(cond[...] * Wp1[...][0:1, :] + bp1[...][0:1, :])
    emb = _pair_norm(jnp.dot(h, Wp2[...], preferred_element_type=jnp.float32)
                     + bp2[...][0:1, :])

    constraints = emb
    variables = jnp.ones((_VAR, _F), dtype=jnp.float32)

    Wc1r = Wc1[...]
    cbias = (jnp.dot(emb, Wc1r[_F:2 * _F, :], preferred_element_type=jnp.float32)
             + bc1[...][0:1, :])

    for i in range(_STEPS):
        acc = [jnp.zeros((_CON, _F), dtype=jnp.float32)]

        def v2c_blk(b, a, variables=variables, acc=acc):
            acc[0] = acc[0] + lax.dot_general(
                a, variables[b * _BLK:(b + 1) * _BLK, :],
                dimension_numbers=(((0,), (0,)), ((), ())),
                preferred_element_type=jnp.float32)
        a_pass(v2c_blk)
        v2c = acc[0]
        hc = _leaky(jnp.dot(constraints, Wc1r[0:_F, :], preferred_element_type=jnp.float32)
                    + jnp.dot(v2c, Wc1r[2 * _F:3 * _F, :], preferred_element_type=jnp.float32)
                    + cbias)
        constraints = _pair_norm(jnp.dot(hc, Wc2[...], preferred_element_type=jnp.float32)
                                 + bc2[...][0:1, :])

        c2v_rows = [None] * _NBLK

        def c2v_blk(b, a, constraints=constraints, c2v_rows=c2v_rows):
            c2v_rows[b] = jnp.dot(a, constraints, preferred_element_type=jnp.float32)
        a_pass(c2v_blk)
        c2v = jnp.concatenate(c2v_rows, axis=0)
        hv = _leaky(jnp.dot(variables, Wv1[...][0:_F, :], preferred_element_type=jnp.float32)
                    + jnp.dot(c2v, Wv1[...][_F:2 * _F, :], preferred_element_type=jnp.float32)
                    + bv1[...][0:1, :])
        variables = _pair_norm(jnp.dot(hv, Wv2[...], preferred_element_type=jnp.float32)
                               + bv2[...][0:1, :])

        ho = _leaky(jnp.dot(variables, Wo1[...], preferred_element_type=jnp.float32)
                    + bo1[...][0:1, :])
        out = jnp.sum(ho * Wo2[...][:, 0][None, :], axis=1, keepdims=True) + bo2[...][0, 0]
        logits = out + noise[...][i]
        outs[i][...] = 1.0 / (1.0 + jnp.exp(-logits))


def _message_passing(A, cond2d, noise, weights):
    out_shape = [jax.ShapeDtypeStruct((_VAR, 1), jnp.float32)] * _STEPS
    fn = pl.pallas_call(
        _mp_body,
        in_specs=[pl.BlockSpec(memory_space=pl.ANY)]
                 + [pl.BlockSpec(memory_space=pltpu.VMEM)] * (2 + len(weights)),
        out_specs=[pl.BlockSpec(memory_space=pltpu.VMEM)] * _STEPS,
        out_shape=out_shape,
        scratch_shapes=[pltpu.VMEM((_BLK, _CON), jnp.float32),
                        pltpu.VMEM((_BLK, _CON), jnp.float32),
                        pltpu.SemaphoreType.DMA,
                        pltpu.SemaphoreType.DMA],
    )
    return fn(A, cond2d, noise, *weights)


def kernel(row_idx, col_idx, edge_vals, conditions_values,
           Wp1, bp1, Wp2, bp2, Wc1, bc1, Wc2, bc2,
           Wv1, bv1, Wv2, bv2, Wo1, bo1, Wo2, bo2):
    pad = _NNZ_PAD - row_idx.shape[0]
    row_p = jnp.pad(row_idx.astype(jnp.int32), (0, pad))
    col_p = jnp.pad(col_idx.astype(jnp.int32), (0, pad))
    val_p = jnp.pad(edge_vals, (0, pad))
    A = _sc_densify(row_p, col_p, val_p).reshape(_VAR, _CON)

    nkey = jax.random.key(42)
    noise = jnp.stack([
        3.0 * jax.random.normal(jax.random.fold_in(nkey, i), (_VAR, 1), dtype=jnp.float32)
        for i in range(_STEPS)])

    weights = (Wp1, bp1.reshape(1, _F), Wp2, bp2.reshape(1, _F),
               Wc1, bc1.reshape(1, _F), Wc2, bc2.reshape(1, _F),
               Wv1, bv1.reshape(1, _F), Wv2, bv2.reshape(1, _F),
               Wo1, bo1.reshape(1, _F), Wo2, bo2.reshape(1, 1))
    outs = _message_passing(A, conditions_values.reshape(_CON, 1), noise, weights)
    return tuple(outs)


# speedup vs baseline: 1.9901x; 1.0544x over previous
import functools

import jax
import jax.numpy as jnp
from jax import lax
from jax.experimental import pallas as pl
from jax.experimental.pallas import tpu as pltpu
from jax.experimental.pallas import tpu_sc as plsc

_F = 64
_VAR = 4096
_CON = 4096
_STEPS = 4
_BLK = 256
_NBLK = _VAR // _BLK


_STRIPES_PER_CORE = 8
_STRIPE_ROWS = 256
_STRIPE_WORDS = _STRIPE_ROWS * _CON
_LAST_ROWS = _VAR - (2 * _STRIPES_PER_CORE - 1) * _STRIPE_ROWS
_LAST_WORDS = _LAST_ROWS * _CON
_CHUNK = 4096
_NNZ_PAD = 1703936
_EDGES_PER_TILE = _NNZ_PAD // 16
_CHUNKS_PER_TILE = _EDGES_PER_TILE // _CHUNK
_WB_WORDS = _STRIPE_WORDS // 16
_WBL_WORDS = _LAST_WORDS // 16


def _densify_body(row_hbm, col_hbm, val_hbm, out_hbm,
                  row_b0, col_b0, val_b0, idx_b0, sval_b0,
                  row_b1, col_b1, val_b1, idx_b1, sval_b1,
                  zbuf, stripe_sh, sem_in0, sem_in1, sem_sc0, sem_sc1):
    cid = lax.axis_index("c")
    sid = lax.axis_index("s")
    estart = sid * _EDGES_PER_TILE
    lanes = lax.iota(jnp.int32, 16)
    rowb = (row_b0, row_b1)
    colb = (col_b0, col_b1)
    valb = (val_b0, val_b1)
    idxb = (idx_b0, idx_b1)
    svalb = (sval_b0, sval_b1)
    sem_in = (sem_in0, sem_in1)
    sem_sc = (sem_sc0, sem_sc1)

    def zinit(i, _):
        zbuf[pl.ds(i * 16, 16)] = jnp.zeros((16,), jnp.float32)
        return 0
    lax.fori_loop(0, zbuf.shape[0] // 16, zinit, 0)

    nzero = _STRIPE_WORDS // 16 // zbuf.shape[0]
    ztail = _STRIPE_WORDS // 16 - nzero * zbuf.shape[0]

    for k in range(_STRIPES_PER_CORE):
        stripe = cid * _STRIPES_PER_CORE + k
        base = stripe * _STRIPE_WORDS

        zoff = sid * (_STRIPE_WORDS // 16)

        def zcopy(z, _):
            pltpu.sync_copy(
                zbuf, stripe_sh.at[pl.ds(zoff + z * zbuf.shape[0],
                                         zbuf.shape[0])])
            return 0
        lax.fori_loop(0, nzero, zcopy, 0)
        if ztail:
            pltpu.sync_copy(zbuf.at[pl.ds(0, ztail)],
                            stripe_sh.at[pl.ds(zoff + nzero * zbuf.shape[0],
                                               ztail)])
        plsc.subcore_barrier()

        def start_in(j, s):
            off = estart + j * _CHUNK
            pltpu.async_copy(row_hbm.at[pl.ds(off, _CHUNK)], rowb[s], sem_in[s])
            pltpu.async_copy(col_hbm.at[pl.ds(off, _CHUNK)], colb[s], sem_in[s])
            pltpu.async_copy(val_hbm.at[pl.ds(off, _CHUNK)], valb[s], sem_in[s])

        def wait_in(s):
            for dst in (rowb[s], colb[s], valb[s]):
                pltpu.make_async_copy(row_hbm.at[pl.ds(0, _CHUNK)], dst,
                                      sem_in[s]).wait()

        def wait_sc(s):
            pltpu.make_async_copy(svalb[s], stripe_sh.at[idxb[s]],
                                  sem_sc[s]).wait()

        def process(jj, s):
            def vec_body(i, _):
                r = rowb[s][pl.ds(i * 16, 16)]
                c = colb[s][pl.ds(i * 16, 16)]
                v = valb[s][pl.ds(i * 16, 16)]
                local = r * _CON + c - base
                ok = (local >= 0) & (local < _STRIPE_WORDS)
                junk = sid * _CHUNK + i * 16 + lanes
                idxb[s][pl.ds(i * 16, 16)] = jnp.where(ok, local, junk)
                svalb[s][pl.ds(i * 16, 16)] = jnp.where(ok, v, 0.0)
                return 0
            lax.fori_loop(0, _CHUNK // 16, vec_body, 0)
            pltpu.async_copy(svalb[s], stripe_sh.at[idxb[s]],
                             sem_sc[s], add=True)

        start_in(0, 0)

        def pair_body(p, _):
            for s in (0, 1):
                jj = 2 * p + s

                @pl.when(jj + 1 < _CHUNKS_PER_TILE)
                def _():
                    start_in(jj + 1, (s + 1) % 2)
                wait_in(s)

                @pl.when(jj >= 2)
                def _():
                    wait_sc(s)
                process(jj, s)
            return 0
        lax.fori_loop(0, _CHUNKS_PER_TILE // 2, pair_body, 0)
        wait_sc(0)
        wait_sc(1)
        plsc.subcore_barrier()

        if k < _STRIPES_PER_CORE - 1:
            pltpu.sync_copy(stripe_sh.at[pl.ds(sid * _WB_WORDS, _WB_WORDS)],
                            out_hbm.at[pl.ds(base + sid * _WB_WORDS, _WB_WORDS)])
        else:
            @pl.when(cid == 0)
            def _():
                pltpu.sync_copy(
                    stripe_sh.at[pl.ds(sid * _WB_WORDS, _WB_WORDS)],
                    out_hbm.at[pl.ds(base + sid * _WB_WORDS, _WB_WORDS)])

            @pl.when(cid == 1)
            def _():
                pltpu.sync_copy(
                    stripe_sh.at[pl.ds(sid * _WBL_WORDS, _WBL_WORDS)],
                    out_hbm.at[pl.ds(base + sid * _WBL_WORDS, _WBL_WORDS)])
        plsc.subcore_barrier()


def _sc_densify(row_p, col_p, val_p):
    fn = pl.kernel(
        _densify_body,
        out_type=jax.ShapeDtypeStruct((_VAR * _CON,), jnp.float32),
        mesh=plsc.VectorSubcoreMesh(core_axis_name="c", subcore_axis_name="s"),
        scratch_types=[
            pltpu.VMEM((_CHUNK,), jnp.int32),
            pltpu.VMEM((_CHUNK,), jnp.int32),
            pltpu.VMEM((_CHUNK,), jnp.float32),
            pltpu.VMEM((_CHUNK,), jnp.int32),
            pltpu.VMEM((_CHUNK,), jnp.float32),
            pltpu.VMEM((_CHUNK,), jnp.int32),
            pltpu.VMEM((_CHUNK,), jnp.int32),
            pltpu.VMEM((_CHUNK,), jnp.float32),
            pltpu.VMEM((_CHUNK,), jnp.int32),
            pltpu.VMEM((_CHUNK,), jnp.float32),
            pltpu.VMEM((4096,), jnp.float32),
            pltpu.VMEM_SHARED((_STRIPE_WORDS,), jnp.float32),
            pltpu.SemaphoreType.DMA,
            pltpu.SemaphoreType.DMA,
            pltpu.SemaphoreType.DMA,
            pltpu.SemaphoreType.DMA,
        ],
    )
    return fn(row_p, col_p, val_p)


def _pair_norm(x):
    x = x - jnp.mean(x, axis=0, keepdims=True)
    rownorm_mean = jnp.sqrt(1e-06 + jnp.mean(jnp.sum(x * x, axis=1)))
    return x / rownorm_mean


def _leaky(x):
    return jnp.where(x >= 0, x, 0.01 * x)


def _mp_body(A_hbm, cond, noise,
             Wp1, bp1, Wp2, bp2, Wc1, bc1, Wc2, bc2,
             Wv1, bv1, Wv2, bv2, Wo1, bo1, Wo2, bo2,
             o0, o1, o2, o3, ablk0, ablk1, sem0, sem1):
    outs = (o0, o1, o2, o3)
    bufs = (ablk0, ablk1)
    sems = (sem0, sem1)

    def a_pass(consume):
        pltpu.make_async_copy(A_hbm.at[pl.ds(0, _BLK), :], bufs[0],
                              sems[0]).start()
        for b in range(_NBLK):
            pltpu.make_async_copy(A_hbm.at[pl.ds(b * _BLK, _BLK), :],
                                  bufs[b % 2], sems[b % 2]).wait()
            if b + 1 < _NBLK:
                pltpu.make_async_copy(A_hbm.at[pl.ds((b + 1) * _BLK, _BLK), :],
                                      bufs[(b + 1) % 2], sems[(b + 1) % 2]).start()
            consume(b, bufs[b % 2][...])

    h = _leaky(cond[...] * Wp1[...][0:1, :] + bp1[...][0:1, :])
    emb = _pair_norm(jnp.dot(h, Wp2[...], preferred_element_type=jnp.float32)
                     + bp2[...][0:1, :])

    constraints = emb
    variables = jnp.ones((_VAR, _F), dtype=jnp.float32)

    Wc1r = Wc1[...]
    cbias = (jnp.dot(emb, Wc1r[_F:2 * _F, :], preferred_element_type=jnp.float32)
             + bc1[...][0:1, :])

    for i in range(_STEPS):
        acc = [jnp.zeros((_CON, _F), dtype=jnp.float32)]
        vbf = variables.astype(jnp.bfloat16)

        def v2c_blk(b, a, vbf=vbf, acc=acc):
            acc[0] = acc[0] + lax.dot_general(
                a, vbf[b * _BLK:(b + 1) * _BLK, :],
                dimension_numbers=(((0,), (0,)), ((), ())),
                preferred_element_type=jnp.float32)
        a_pass(v2c_blk)
        v2c = acc[0]
        hc = _leaky(jnp.dot(constraints, Wc1r[0:_F, :], preferred_element_type=jnp.float32)
                    + jnp.dot(v2c, Wc1r[2 * _F:3 * _F, :], preferred_element_type=jnp.float32)
                    + cbias)
        constraints = _pair_norm(jnp.dot(hc, Wc2[...], preferred_element_type=jnp.float32)
                                 + bc2[...][0:1, :])

        c2v_rows = [None] * _NBLK
        cbf = constraints.astype(jnp.bfloat16)

        def c2v_blk(b, a, cbf=cbf, c2v_rows=c2v_rows):
            c2v_rows[b] = jnp.dot(a, cbf, preferred_element_type=jnp.float32)
        a_pass(c2v_blk)
        c2v = jnp.concatenate(c2v_rows, axis=0)
        hv = _leaky(jnp.dot(variables, Wv1[...][0:_F, :], preferred_element_type=jnp.float32)
                    + jnp.dot(c2v, Wv1[...][_F:2 * _F, :], preferred_element_type=jnp.float32)
                    + bv1[...][0:1, :])
        variables = _pair_norm(jnp.dot(hv, Wv2[...], preferred_element_type=jnp.float32)
                               + bv2[...][0:1, :])

        ho = _leaky(jnp.dot(variables, Wo1[...], preferred_element_type=jnp.float32)
                    + bo1[...][0:1, :])
        out = jnp.sum(ho * Wo2[...][:, 0][None, :], axis=1, keepdims=True) + bo2[...][0, 0]
        logits = out + noise[...][i]
        outs[i][...] = 1.0 / (1.0 + jnp.exp(-logits))


def _to_bf16(A):
    return pl.pallas_call(
        lambda a_ref, o_ref: o_ref.__setitem__(
            (Ellipsis,), a_ref[...].astype(jnp.bfloat16)),
        grid=(_NBLK,),
        in_specs=[pl.BlockSpec((_BLK, _CON), lambda b: (b, 0))],
        out_specs=pl.BlockSpec((_BLK, _CON), lambda b: (b, 0)),
        out_shape=jax.ShapeDtypeStruct((_VAR, _CON), jnp.bfloat16),
    )(A)


def _message_passing(A, cond2d, noise, weights):
    out_shape = [jax.ShapeDtypeStruct((_VAR, 1), jnp.float32)] * _STEPS
    fn = pl.pallas_call(
        _mp_body,
        in_specs=[pl.BlockSpec(memory_space=pl.ANY)]
                 + [pl.BlockSpec(memory_space=pltpu.VMEM)] * (2 + len(weights)),
        out_specs=[pl.BlockSpec(memory_space=pltpu.VMEM)] * _STEPS,
        out_shape=out_shape,
        scratch_shapes=[pltpu.VMEM((_BLK, _CON), jnp.bfloat16),
                        pltpu.VMEM((_BLK, _CON), jnp.bfloat16),
                        pltpu.SemaphoreType.DMA,
                        pltpu.SemaphoreType.DMA],
    )
    return fn(A, cond2d, noise, *weights)


def kernel(row_idx, col_idx, edge_vals, conditions_values,
           Wp1, bp1, Wp2, bp2, Wc1, bc1, Wc2, bc2,
           Wv1, bv1, Wv2, bv2, Wo1, bo1, Wo2, bo2):
    pad = _NNZ_PAD - row_idx.shape[0]
    row_p = jnp.pad(row_idx.astype(jnp.int32), (0, pad))
    col_p = jnp.pad(col_idx.astype(jnp.int32), (0, pad))
    val_p = jnp.pad(edge_vals, (0, pad))
    A = _to_bf16(_sc_densify(row_p, col_p, val_p).reshape(_VAR, _CON))

    nkey = jax.random.key(42)
    noise = jnp.stack([
        3.0 * jax.random.normal(jax.random.fold_in(nkey, i), (_VAR, 1), dtype=jnp.float32)
        for i in range(_STEPS)])

    weights = (Wp1, bp1.reshape(1, _F), Wp2, bp2.reshape(1, _F),
               Wc1, bc1.reshape(1, _F), Wc2, bc2.reshape(1, _F),
               Wv1, bv1.reshape(1, _F), Wv2, bv2.reshape(1, _F),
               Wo1, bo1.reshape(1, _F), Wo2, bo2.reshape(1, 1))
    outs = _message_passing(A, conditions_values.reshape(_CON, 1), noise, weights)
    return tuple(outs)


# speedup vs baseline: 2.1136x; 1.0620x over previous
import functools

import jax
import jax.numpy as jnp
from jax import lax
from jax.experimental import pallas as pl
from jax.experimental.pallas import tpu as pltpu
from jax.experimental.pallas import tpu_sc as plsc

_F = 64
_VAR = 4096
_CON = 4096
_STEPS = 4
_BLK = 512
_NBLK = _VAR // _BLK


_STRIPES_PER_CORE = 8
_STRIPE_ROWS = 256
_STRIPE_WORDS = _STRIPE_ROWS * _CON
_LAST_ROWS = _VAR - (2 * _STRIPES_PER_CORE - 1) * _STRIPE_ROWS
_LAST_WORDS = _LAST_ROWS * _CON
_CHUNK = 4096
_NNZ_PAD = 1703936
_EDGES_PER_TILE = _NNZ_PAD // 16
_CHUNKS_PER_TILE = _EDGES_PER_TILE // _CHUNK
_WB_WORDS = _STRIPE_WORDS // 16
_WBL_WORDS = _LAST_WORDS // 16


def _densify_body(row_hbm, col_hbm, val_hbm, out_hbm,
                  row_b0, col_b0, val_b0, idx_b0, sval_b0,
                  row_b1, col_b1, val_b1, idx_b1, sval_b1,
                  zbuf, stripe_sh, sem_in0, sem_in1, sem_sc0, sem_sc1):
    cid = lax.axis_index("c")
    sid = lax.axis_index("s")
    estart = sid * _EDGES_PER_TILE
    lanes = lax.iota(jnp.int32, 16)
    rowb = (row_b0, row_b1)
    colb = (col_b0, col_b1)
    valb = (val_b0, val_b1)
    idxb = (idx_b0, idx_b1)
    svalb = (sval_b0, sval_b1)
    sem_in = (sem_in0, sem_in1)
    sem_sc = (sem_sc0, sem_sc1)

    def zinit(i, _):
        zbuf[pl.ds(i * 16, 16)] = jnp.zeros((16,), jnp.float32)
        return 0
    lax.fori_loop(0, zbuf.shape[0] // 16, zinit, 0)

    nzero = _STRIPE_WORDS // 16 // zbuf.shape[0]
    ztail = _STRIPE_WORDS // 16 - nzero * zbuf.shape[0]

    for k in range(_STRIPES_PER_CORE):
        stripe = cid * _STRIPES_PER_CORE + k
        base = stripe * _STRIPE_WORDS

        zoff = sid * (_STRIPE_WORDS // 16)

        def zcopy(z, _):
            pltpu.sync_copy(
                zbuf, stripe_sh.at[pl.ds(zoff + z * zbuf.shape[0],
                                         zbuf.shape[0])])
            return 0
        lax.fori_loop(0, nzero, zcopy, 0)
        if ztail:
            pltpu.sync_copy(zbuf.at[pl.ds(0, ztail)],
                            stripe_sh.at[pl.ds(zoff + nzero * zbuf.shape[0],
                                               ztail)])
        plsc.subcore_barrier()

        def start_in(j, s):
            off = estart + j * _CHUNK
            pltpu.async_copy(row_hbm.at[pl.ds(off, _CHUNK)], rowb[s], sem_in[s])
            pltpu.async_copy(col_hbm.at[pl.ds(off, _CHUNK)], colb[s], sem_in[s])
            pltpu.async_copy(val_hbm.at[pl.ds(off, _CHUNK)], valb[s], sem_in[s])

        def wait_in(s):
            for dst in (rowb[s], colb[s], valb[s]):
                pltpu.make_async_copy(row_hbm.at[pl.ds(0, _CHUNK)], dst,
                                      sem_in[s]).wait()

        def wait_sc(s):
            pltpu.make_async_copy(svalb[s], stripe_sh.at[idxb[s]],
                                  sem_sc[s]).wait()

        def process(jj, s):
            def vec_body(i, _):
                r = rowb[s][pl.ds(i * 16, 16)]
                c = colb[s][pl.ds(i * 16, 16)]
                v = valb[s][pl.ds(i * 16, 16)]
                local = r * _CON + c - base
                ok = (local >= 0) & (local < _STRIPE_WORDS)
                junk = sid * _CHUNK + i * 16 + lanes
                idxb[s][pl.ds(i * 16, 16)] = jnp.where(ok, local, junk)
                svalb[s][pl.ds(i * 16, 16)] = jnp.where(ok, v, 0.0)
                return 0
            lax.fori_loop(0, _CHUNK // 16, vec_body, 0)
            pltpu.async_copy(svalb[s], stripe_sh.at[idxb[s]],
                             sem_sc[s], add=True)

        start_in(0, 0)

        def pair_body(p, _):
            for s in (0, 1):
                jj = 2 * p + s

                @pl.when(jj + 1 < _CHUNKS_PER_TILE)
                def _():
                    start_in(jj + 1, (s + 1) % 2)
                wait_in(s)

                @pl.when(jj >= 2)
                def _():
                    wait_sc(s)
                process(jj, s)
            return 0
        lax.fori_loop(0, _CHUNKS_PER_TILE // 2, pair_body, 0)
        wait_sc(0)
        wait_sc(1)
        plsc.subcore_barrier()

        if k < _STRIPES_PER_CORE - 1:
            pltpu.sync_copy(stripe_sh.at[pl.ds(sid * _WB_WORDS, _WB_WORDS)],
                            out_hbm.at[pl.ds(base + sid * _WB_WORDS, _WB_WORDS)])
        else:
            @pl.when(cid == 0)
            def _():
                pltpu.sync_copy(
                    stripe_sh.at[pl.ds(sid * _WB_WORDS, _WB_WORDS)],
                    out_hbm.at[pl.ds(base + sid * _WB_WORDS, _WB_WORDS)])

            @pl.when(cid == 1)
            def _():
                pltpu.sync_copy(
                    stripe_sh.at[pl.ds(sid * _WBL_WORDS, _WBL_WORDS)],
                    out_hbm.at[pl.ds(base + sid * _WBL_WORDS, _WBL_WORDS)])
        plsc.subcore_barrier()


def _sc_densify(row_p, col_p, val_p):
    fn = pl.kernel(
        _densify_body,
        out_type=jax.ShapeDtypeStruct((_VAR * _CON,), jnp.float32),
        mesh=plsc.VectorSubcoreMesh(core_axis_name="c", subcore_axis_name="s"),
        scratch_types=[
            pltpu.VMEM((_CHUNK,), jnp.int32),
            pltpu.VMEM((_CHUNK,), jnp.int32),
            pltpu.VMEM((_CHUNK,), jnp.float32),
            pltpu.VMEM((_CHUNK,), jnp.int32),
            pltpu.VMEM((_CHUNK,), jnp.float32),
            pltpu.VMEM((_CHUNK,), jnp.int32),
            pltpu.VMEM((_CHUNK,), jnp.int32),
            pltpu.VMEM((_CHUNK,), jnp.float32),
            pltpu.VMEM((_CHUNK,), jnp.int32),
            pltpu.VMEM((_CHUNK,), jnp.float32),
            pltpu.VMEM((4096,), jnp.float32),
            pltpu.VMEM_SHARED((_STRIPE_WORDS,), jnp.float32),
            pltpu.SemaphoreType.DMA,
            pltpu.SemaphoreType.DMA,
            pltpu.SemaphoreType.DMA,
            pltpu.SemaphoreType.DMA,
        ],
    )
    return fn(row_p, col_p, val_p)


def _pair_norm(x):
    x = x - jnp.mean(x, axis=0, keepdims=True)
    rownorm_mean = jnp.sqrt(1e-06 + jnp.mean(jnp.sum(x * x, axis=1)))
    return x / rownorm_mean


def _leaky(x):
    return jnp.where(x >= 0, x, 0.01 * x)


def _mp_body(A_hbm, cond, noise,
             Wp1, bp1, Wp2, bp2, Wc1, bc1, Wc2, bc2,
             Wv1, bv1, Wv2, bv2, Wo1, bo1, Wo2, bo2,
             o0, o1, o2, o3, ablk0, ablk1, sem0, sem1):
    outs = (o0, o1, o2, o3)
    bufs = (ablk0, ablk1)
    sems = (sem0, sem1)

    def a_pass(consume):
        pltpu.make_async_copy(A_hbm.at[pl.ds(0, _BLK), :], bufs[0],
                              sems[0]).start()
        for b in range(_NBLK):
            pltpu.make_async_copy(A_hbm.at[pl.ds(b * _BLK, _BLK), :],
                                  bufs[b % 2], sems[b % 2]).wait()
            if b + 1 < _NBLK:
                pltpu.make_async_copy(A_hbm.at[pl.ds((b + 1) * _BLK, _BLK), :],
                                      bufs[(b + 1) % 2], sems[(b + 1) % 2]).start()
            consume(b, bufs[b % 2][...])

    h = _leaky(cond[...] * Wp1[...][0:1, :] + bp1[...][0:1, :])
    emb = _pair_norm(jnp.dot(h, Wp2[...], preferred_element_type=jnp.float32)
                     + bp2[...][0:1, :])

    constraints = emb
    variables = jnp.ones((_VAR, _F), dtype=jnp.float32)

    Wc1r = Wc1[...]
    cbias = (jnp.dot(emb, Wc1r[_F:2 * _F, :], preferred_element_type=jnp.float32)
             + bc1[...][0:1, :])

    for i in range(_STEPS):
        acc = [jnp.zeros((_CON, _F), dtype=jnp.float32)]
        vbf = variables.astype(jnp.bfloat16)

        def v2c_blk(b, a, vbf=vbf, acc=acc):
            acc[0] = acc[0] + lax.dot_general(
                a, vbf[b * _BLK:(b + 1) * _BLK, :],
                dimension_numbers=(((0,), (0,)), ((), ())),
                preferred_element_type=jnp.float32)
        a_pass(v2c_blk)
        v2c = acc[0]
        hc = _leaky(jnp.dot(constraints, Wc1r[0:_F, :], preferred_element_type=jnp.float32)
                    + jnp.dot(v2c, Wc1r[2 * _F:3 * _F, :], preferred_element_type=jnp.float32)
                    + cbias)
        constraints = _pair_norm(jnp.dot(hc, Wc2[...], preferred_element_type=jnp.float32)
                                 + bc2[...][0:1, :])

        c2v_rows = [None] * _NBLK
        cbf = constraints.astype(jnp.bfloat16)

        def c2v_blk(b, a, cbf=cbf, c2v_rows=c2v_rows):
            c2v_rows[b] = jnp.dot(a, cbf, preferred_element_type=jnp.float32)
        a_pass(c2v_blk)
        c2v = jnp.concatenate(c2v_rows, axis=0)
        hv = _leaky(jnp.dot(variables, Wv1[...][0:_F, :], preferred_element_type=jnp.float32)
                    + jnp.dot(c2v, Wv1[...][_F:2 * _F, :], preferred_element_type=jnp.float32)
                    + bv1[...][0:1, :])
        variables = _pair_norm(jnp.dot(hv, Wv2[...], preferred_element_type=jnp.float32)
                               + bv2[...][0:1, :])

        ho = _leaky(jnp.dot(variables, Wo1[...], preferred_element_type=jnp.float32)
                    + bo1[...][0:1, :])
        out = jnp.sum(ho * Wo2[...][:, 0][None, :], axis=1, keepdims=True) + bo2[...][0, 0]
        logits = out + noise[...][i]
        outs[i][...] = 1.0 / (1.0 + jnp.exp(-logits))


def _to_bf16(A):
    return pl.pallas_call(
        lambda a_ref, o_ref: o_ref.__setitem__(
            (Ellipsis,), a_ref[...].astype(jnp.bfloat16)),
        grid=(_NBLK,),
        in_specs=[pl.BlockSpec((_BLK, _CON), lambda b: (b, 0))],
        out_specs=pl.BlockSpec((_BLK, _CON), lambda b: (b, 0)),
        out_shape=jax.ShapeDtypeStruct((_VAR, _CON), jnp.bfloat16),
    )(A)


def _message_passing(A, cond2d, noise, weights):
    out_shape = [jax.ShapeDtypeStruct((_VAR, 1), jnp.float32)] * _STEPS
    fn = pl.pallas_call(
        _mp_body,
        in_specs=[pl.BlockSpec(memory_space=pl.ANY)]
                 + [pl.BlockSpec(memory_space=pltpu.VMEM)] * (2 + len(weights)),
        out_specs=[pl.BlockSpec(memory_space=pltpu.VMEM)] * _STEPS,
        out_shape=out_shape,
        scratch_shapes=[pltpu.VMEM((_BLK, _CON), jnp.bfloat16),
                        pltpu.VMEM((_BLK, _CON), jnp.bfloat16),
                        pltpu.SemaphoreType.DMA,
                        pltpu.SemaphoreType.DMA],
    )
    return fn(A, cond2d, noise, *weights)


def kernel(row_idx, col_idx, edge_vals, conditions_values,
           Wp1, bp1, Wp2, bp2, Wc1, bc1, Wc2, bc2,
           Wv1, bv1, Wv2, bv2, Wo1, bo1, Wo2, bo2):
    pad = _NNZ_PAD - row_idx.shape[0]
    row_p = jnp.pad(row_idx.astype(jnp.int32), (0, pad))
    col_p = jnp.pad(col_idx.astype(jnp.int32), (0, pad))
    val_p = jnp.pad(edge_vals, (0, pad))
    A = _to_bf16(_sc_densify(row_p, col_p, val_p).reshape(_VAR, _CON))

    nkey = jax.random.key(42)
    noise = jnp.stack([
        3.0 * jax.random.normal(jax.random.fold_in(nkey, i), (_VAR, 1), dtype=jnp.float32)
        for i in range(_STEPS)])

    weights = (Wp1, bp1.reshape(1, _F), Wp2, bp2.reshape(1, _F),
               Wc1, bc1.reshape(1, _F), Wc2, bc2.reshape(1, _F),
               Wv1, bv1.reshape(1, _F), Wv2, bv2.reshape(1, _F),
               Wo1, bo1.reshape(1, _F), Wo2, bo2.reshape(1, 1))
    outs = _message_passing(A, conditions_values.reshape(_CON, 1), noise, weights)
    return tuple(outs)


# speedup vs baseline: 2.1313x; 1.0084x over previous
import functools

import jax
import jax.numpy as jnp
from jax import lax
from jax.experimental import pallas as pl
from jax.experimental.pallas import tpu as pltpu
from jax.experimental.pallas import tpu_sc as plsc

_F = 64
_VAR = 4096
_CON = 4096
_STEPS = 4
_BLK = 512
_NBLK = _VAR // _BLK


_STRIPES_PER_CORE = 8
_STRIPE_ROWS = 256
_STRIPE_WORDS = _STRIPE_ROWS * _CON
_LAST_ROWS = _VAR - (2 * _STRIPES_PER_CORE - 1) * _STRIPE_ROWS
_LAST_WORDS = _LAST_ROWS * _CON
_CHUNK = 4096
_NNZ_PAD = 1703936
_EDGES_PER_TILE = _NNZ_PAD // 16
_CHUNKS_PER_TILE = _EDGES_PER_TILE // _CHUNK
_WB_WORDS = _STRIPE_WORDS // 16
_WBL_WORDS = _LAST_WORDS // 16


def _densify_body(flat_hbm, val_hbm, out_hbm,
                  flat_b0, val_b0, idx_b0, sval_b0,
                  flat_b1, val_b1, idx_b1, sval_b1,
                  zbuf, stripe_sh, sem_in0, sem_in1, sem_sc0, sem_sc1):
    cid = lax.axis_index("c")
    sid = lax.axis_index("s")
    estart = sid * _EDGES_PER_TILE
    lanes = lax.iota(jnp.int32, 16)
    flatb = (flat_b0, flat_b1)
    valb = (val_b0, val_b1)
    idxb = (idx_b0, idx_b1)
    svalb = (sval_b0, sval_b1)
    sem_in = (sem_in0, sem_in1)
    sem_sc = (sem_sc0, sem_sc1)

    def zinit(i, _):
        zbuf[pl.ds(i * 16, 16)] = jnp.zeros((16,), jnp.float32)
        return 0
    lax.fori_loop(0, zbuf.shape[0] // 16, zinit, 0)

    nzero = _STRIPE_WORDS // 16 // zbuf.shape[0]
    ztail = _STRIPE_WORDS // 16 - nzero * zbuf.shape[0]

    for k in range(_STRIPES_PER_CORE):
        stripe = cid * _STRIPES_PER_CORE + k
        base = stripe * _STRIPE_WORDS

        zoff = sid * (_STRIPE_WORDS // 16)

        def zcopy(z, _):
            pltpu.sync_copy(
                zbuf, stripe_sh.at[pl.ds(zoff + z * zbuf.shape[0],
                                         zbuf.shape[0])])
            return 0
        lax.fori_loop(0, nzero, zcopy, 0)
        if ztail:
            pltpu.sync_copy(zbuf.at[pl.ds(0, ztail)],
                            stripe_sh.at[pl.ds(zoff + nzero * zbuf.shape[0],
                                               ztail)])
        plsc.subcore_barrier()

        def start_in(j, s):
            off = estart + j * _CHUNK
            pltpu.async_copy(flat_hbm.at[pl.ds(off, _CHUNK)], flatb[s], sem_in[s])
            pltpu.async_copy(val_hbm.at[pl.ds(off, _CHUNK)], valb[s], sem_in[s])

        def wait_in(s):
            for dst in (flatb[s], valb[s]):
                pltpu.make_async_copy(flat_hbm.at[pl.ds(0, _CHUNK)], dst,
                                      sem_in[s]).wait()

        def wait_sc(s):
            pltpu.make_async_copy(svalb[s], stripe_sh.at[idxb[s]],
                                  sem_sc[s]).wait()

        def process(jj, s):
            def vec_body(i, _):
                f = flatb[s][pl.ds(i * 16, 16)]
                v = valb[s][pl.ds(i * 16, 16)]
                local = f - base
                ok = (local >= 0) & (local < _STRIPE_WORDS)
                junk = sid * _CHUNK + i * 16 + lanes
                idxb[s][pl.ds(i * 16, 16)] = jnp.where(ok, local, junk)
                svalb[s][pl.ds(i * 16, 16)] = jnp.where(ok, v, 0.0)
                return 0
            lax.fori_loop(0, _CHUNK // 16, vec_body, 0)
            pltpu.async_copy(svalb[s], stripe_sh.at[idxb[s]],
                             sem_sc[s], add=True)

        start_in(0, 0)

        def pair_body(p, _):
            for s in (0, 1):
                jj = 2 * p + s

                @pl.when(jj + 1 < _CHUNKS_PER_TILE)
                def _():
                    start_in(jj + 1, (s + 1) % 2)
                wait_in(s)

                @pl.when(jj >= 2)
                def _():
                    wait_sc(s)
                process(jj, s)
            return 0
        lax.fori_loop(0, _CHUNKS_PER_TILE // 2, pair_body, 0)
        wait_sc(0)
        wait_sc(1)
        plsc.subcore_barrier()

        if k < _STRIPES_PER_CORE - 1:
            pltpu.sync_copy(stripe_sh.at[pl.ds(sid * _WB_WORDS, _WB_WORDS)],
                            out_hbm.at[pl.ds(base + sid * _WB_WORDS, _WB_WORDS)])
        else:
            @pl.when(cid == 0)
            def _():
                pltpu.sync_copy(
                    stripe_sh.at[pl.ds(sid * _WB_WORDS, _WB_WORDS)],
                    out_hbm.at[pl.ds(base + sid * _WB_WORDS, _WB_WORDS)])

            @pl.when(cid == 1)
            def _():
                pltpu.sync_copy(
                    stripe_sh.at[pl.ds(sid * _WBL_WORDS, _WBL_WORDS)],
                    out_hbm.at[pl.ds(base + sid * _WBL_WORDS, _WBL_WORDS)])
        plsc.subcore_barrier()


def _sc_densify(flat_p, val_p):
    fn = pl.kernel(
        _densify_body,
        out_type=jax.ShapeDtypeStruct((_VAR * _CON,), jnp.float32),
        mesh=plsc.VectorSubcoreMesh(core_axis_name="c", subcore_axis_name="s"),
        scratch_types=[
            pltpu.VMEM((_CHUNK,), jnp.int32),
            pltpu.VMEM((_CHUNK,), jnp.float32),
            pltpu.VMEM((_CHUNK,), jnp.int32),
            pltpu.VMEM((_CHUNK,), jnp.float32),
            pltpu.VMEM((_CHUNK,), jnp.int32),
            pltpu.VMEM((_CHUNK,), jnp.float32),
            pltpu.VMEM((_CHUNK,), jnp.int32),
            pltpu.VMEM((_CHUNK,), jnp.float32),
            pltpu.VMEM((4096,), jnp.float32),
            pltpu.VMEM_SHARED((_STRIPE_WORDS,), jnp.float32),
            pltpu.SemaphoreType.DMA,
            pltpu.SemaphoreType.DMA,
            pltpu.SemaphoreType.DMA,
            pltpu.SemaphoreType.DMA,
        ],
    )
    return fn(flat_p, val_p)


def _pair_norm(x):
    x = x - jnp.mean(x, axis=0, keepdims=True)
    rownorm_mean = jnp.sqrt(1e-06 + jnp.mean(jnp.sum(x * x, axis=1)))
    return x / rownorm_mean


def _leaky(x):
    return jnp.where(x >= 0, x, 0.01 * x)


def _mp_body(A_hbm, cond, noise,
             Wp1, bp1, Wp2, bp2, Wc1, bc1, Wc2, bc2,
             Wv1, bv1, Wv2, bv2, Wo1, bo1, Wo2, bo2,
             o0, o1, o2, o3, ablk0, ablk1, sem0, sem1):
    outs = (o0, o1, o2, o3)
    bufs = (ablk0, ablk1)
    sems = (sem0, sem1)

    def a_pass(consume):
        pltpu.make_async_copy(A_hbm.at[pl.ds(0, _BLK), :], bufs[0],
                              sems[0]).start()
        for b in range(_NBLK):
            pltpu.make_async_copy(A_hbm.at[pl.ds(b * _BLK, _BLK), :],
                                  bufs[b % 2], sems[b % 2]).wait()
            if b + 1 < _NBLK:
                pltpu.make_async_copy(A_hbm.at[pl.ds((b + 1) * _BLK, _BLK), :],
                                      bufs[(b + 1) % 2], sems[(b + 1) % 2]).start()
            consume(b, bufs[b % 2][...])

    h = _leaky(cond[...] * Wp1[...][0:1, :] + bp1[...][0:1, :])
    emb = _pair_norm(jnp.dot(h, Wp2[...], preferred_element_type=jnp.float32)
                     + bp2[...][0:1, :])

    constraints = emb
    variables = jnp.ones((_VAR, _F), dtype=jnp.float32)

    Wc1r = Wc1[...]
    cbias = (jnp.dot(emb, Wc1r[_F:2 * _F, :], preferred_element_type=jnp.float32)
             + bc1[...][0:1, :])

    for i in range(_STEPS):
        acc = [jnp.zeros((_CON, _F), dtype=jnp.float32)]
        vbf = variables.astype(jnp.bfloat16)

        def v2c_blk(b, a, vbf=vbf, acc=acc):
            acc[0] = acc[0] + lax.dot_general(
                a, vbf[b * _BLK:(b + 1) * _BLK, :],
                dimension_numbers=(((0,), (0,)), ((), ())),
                preferred_element_type=jnp.float32)
        a_pass(v2c_blk)
        v2c = acc[0]
        hc = _leaky(jnp.dot(constraints, Wc1r[0:_F, :], preferred_element_type=jnp.float32)
                    + jnp.dot(v2c, Wc1r[2 * _F:3 * _F, :], preferred_element_type=jnp.float32)
                    + cbias)
        constraints = _pair_norm(jnp.dot(hc, Wc2[...], preferred_element_type=jnp.float32)
                                 + bc2[...][0:1, :])

        c2v_rows = [None] * _NBLK
        cbf = constraints.astype(jnp.bfloat16)

        def c2v_blk(b, a, cbf=cbf, c2v_rows=c2v_rows):
            c2v_rows[b] = jnp.dot(a, cbf, preferred_element_type=jnp.float32)
        a_pass(c2v_blk)
        c2v = jnp.concatenate(c2v_rows, axis=0)
        hv = _leaky(jnp.dot(variables, Wv1[...][0:_F, :], preferred_element_type=jnp.float32)
                    + jnp.dot(c2v, Wv1[...][_F:2 * _F, :], preferred_element_type=jnp.float32)
                    + bv1[...][0:1, :])
        variables = _pair_norm(jnp.dot(hv, Wv2[...], preferred_element_type=jnp.float32)
                               + bv2[...][0:1, :])

        ho = _leaky(jnp.dot(variables, Wo1[...], preferred_element_type=jnp.float32)
                    + bo1[...][0:1, :])
        out = jnp.sum(ho * Wo2[...][:, 0][None, :], axis=1, keepdims=True) + bo2[...][0, 0]
        logits = out + noise[...][i]
        outs[i][...] = 1.0 / (1.0 + jnp.exp(-logits))


def _to_bf16(A):
    return pl.pallas_call(
        lambda a_ref, o_ref: o_ref.__setitem__(
            (Ellipsis,), a_ref[...].astype(jnp.bfloat16)),
        grid=(_NBLK,),
        in_specs=[pl.BlockSpec((_BLK, _CON), lambda b: (b, 0))],
        out_specs=pl.BlockSpec((_BLK, _CON), lambda b: (b, 0)),
        out_shape=jax.ShapeDtypeStruct((_VAR, _CON), jnp.bfloat16),
    )(A)


def _message_passing(A, cond2d, noise, weights):
    out_shape = [jax.ShapeDtypeStruct((_VAR, 1), jnp.float32)] * _STEPS
    fn = pl.pallas_call(
        _mp_body,
        in_specs=[pl.BlockSpec(memory_space=pl.ANY)]
                 + [pl.BlockSpec(memory_space=pltpu.VMEM)] * (2 + len(weights)),
        out_specs=[pl.BlockSpec(memory_space=pltpu.VMEM)] * _STEPS,
        out_shape=out_shape,
        scratch_shapes=[pltpu.VMEM((_BLK, _CON), jnp.bfloat16),
                        pltpu.VMEM((_BLK, _CON), jnp.bfloat16),
                        pltpu.SemaphoreType.DMA,
                        pltpu.SemaphoreType.DMA],
    )
    return fn(A, cond2d, noise, *weights)


def kernel(row_idx, col_idx, edge_vals, conditions_values,
           Wp1, bp1, Wp2, bp2, Wc1, bc1, Wc2, bc2,
           Wv1, bv1, Wv2, bv2, Wo1, bo1, Wo2, bo2):
    pad = _NNZ_PAD - row_idx.shape[0]
    flat = row_idx.astype(jnp.int32) * _CON + col_idx.astype(jnp.int32)
    flat_p = jnp.pad(flat, (0, pad))
    val_p = jnp.pad(edge_vals, (0, pad))
    A = _to_bf16(_sc_densify(flat_p, val_p).reshape(_VAR, _CON))

    nkey = jax.random.key(42)
    noise = jnp.stack([
        3.0 * jax.random.normal(jax.random.fold_in(nkey, i), (_VAR, 1), dtype=jnp.float32)
        for i in range(_STEPS)])

    weights = (Wp1, bp1.reshape(1, _F), Wp2, bp2.reshape(1, _F),
               Wc1, bc1.reshape(1, _F), Wc2, bc2.reshape(1, _F),
               Wv1, bv1.reshape(1, _F), Wv2, bv2.reshape(1, _F),
               Wo1, bo1.reshape(1, _F), Wo2, bo2.reshape(1, 1))
    outs = _message_passing(A, conditions_values.reshape(_CON, 1), noise, weights)
    return tuple(outs)
